# baseline (speedup 1.0000x reference)
import functools

import jax
import jax.numpy as jnp
from jax import lax
from jax.experimental import pallas as pl
from jax.experimental.pallas import tpu as pltpu
from jax.experimental.pallas import tpu_sc as plsc

BATCH = 16384
OP_D = 24
SHAPE_D = 8
OUT_D = 32
N_SHAPE_VALS = 2


def _body(idx_hbm, sv_hbm, table_hbm, wb_hbm, out_hbm,
          idx_v, sv_v, rows_v, s8_v, wb_v, gsem):
    info = plsc.get_sparse_core_info()
    nc, ns, L = info.num_cores, info.num_subcores, info.num_lanes
    nw = nc * ns
    bpw = BATCH // nw
    wid = lax.axis_index("s") * nc + lax.axis_index("c")
    base = wid * bpw

    nchunk = bpw // 128
    pltpu.sync_copy(idx_hbm.at[pl.ds(wid * nchunk, nchunk), :], idx_v)
    pltpu.sync_copy(sv_hbm.at[pl.ds(base, bpw), :], sv_v)
    pltpu.sync_copy(wb_hbm, wb_v)

    gathers = [
        pltpu.async_copy(table_hbm.at[idx_v.at[j]],
                         rows_v.at[pl.ds(j * 128, 128), :], gsem)
        for j in range(nchunk)
    ]

    lane = lax.iota(jnp.int32, L)
    w_bc = [plsc.load_gather(wb_v, [jnp.full((L,), 1 + k, jnp.int32)])
            for k in range(N_SHAPE_VALS * SHAPE_D)]
    b_bc = [plsc.load_gather(wb_v, [jnp.full((L,), 17 + j, jnp.int32)])
            for j in range(SHAPE_D)]
    zero = jnp.zeros((L,), jnp.int32)

    def chunk(i, carry):
        r = i * L + lane
        sv0 = plsc.load_gather(sv_v, [r, zero])
        sv1 = plsc.load_gather(sv_v, [r, zero + 1])
        for j in range(SHAPE_D):
            col = sv0 * w_bc[j] + sv1 * w_bc[SHAPE_D + j] + b_bc[j]
            plsc.store_scatter(s8_v, [r, jnp.full((L,), j, jnp.int32)], col)
        return carry

    lax.fori_loop(0, bpw // L, chunk, 0)

    for g in gathers:
        g.wait()
    pltpu.sync_copy(rows_v, out_hbm.at[pl.ds(base, bpw), pl.ds(0, OP_D)])
    pltpu.sync_copy(s8_v, out_hbm.at[pl.ds(base, bpw), pl.ds(OP_D, SHAPE_D)])


def kernel(node_inds, shape_vals, op_table, lin_W, lin_b):
    info = plsc.get_sparse_core_info()
    nw = info.num_cores * info.num_subcores
    bpw = BATCH // nw
    mesh = plsc.VectorSubcoreMesh(core_axis_name="c", subcore_axis_name="s")
    k = functools.partial(
        pl.kernel,
        mesh=mesh,
        compiler_params=pltpu.CompilerParams(
            use_tc_tiling_on_sc=False, needs_layout_passes=False),
        out_type=jax.ShapeDtypeStruct((BATCH, OUT_D), jnp.float32),
        scratch_types=[
            pltpu.VMEM((bpw // 128, 128), jnp.int32),
            pltpu.VMEM((bpw, N_SHAPE_VALS), jnp.float32),
            pltpu.VMEM((bpw, OP_D), jnp.float32),
            pltpu.VMEM((bpw, SHAPE_D), jnp.float32),
            pltpu.VMEM((32,), jnp.float32),
            pltpu.SemaphoreType.DMA,
        ],
    )(_body)
    wb = jnp.concatenate(
        [jnp.zeros((1,), jnp.float32), lin_W.reshape(-1), lin_b,
         jnp.zeros((7,), jnp.float32)])
    return k(node_inds.astype(jnp.int32).reshape(-1, 128), shape_vals,
             op_table, wb)
